# baseline (speedup 1.0000x reference)
import jax
import jax.numpy as jnp
from jax.experimental import pallas as pl
from jax.experimental.pallas import tpu as pltpu


def _body(x_ref, pe_ref, o_ref):
    o_ref[...] = x_ref[...] + pe_ref[...]


def kernel(x, pe):
    B, S, D = x.shape
    BS = 512
    xf = x.reshape(B * S, D)
    nseq = S // BS

    part1 = pl.pallas_call(
        _body,
        grid=(nseq, B - 1),
        in_specs=[
            pl.BlockSpec((BS, D), lambda i, b: (b * 4 + i, 0)),
            pl.BlockSpec((BS, D), lambda i, b: (i, 0)),
        ],
        out_specs=pl.BlockSpec((BS, D), lambda i, b: (b * 4 + i, 0)),
        out_shape=jax.ShapeDtypeStruct(((B - 1) * S, D), x.dtype),
    )(xf, pe)

    part2 = pl.pallas_call(
        _body,
        grid=(nseq,),
        in_specs=[
            pl.BlockSpec((BS, D), lambda i: ((B - 1) * 4 + i, 0)),
            pl.BlockSpec((BS, D), lambda i: (i, 0)),
        ],
        out_specs=pl.BlockSpec((BS, D), lambda i: (i, 0)),
        out_shape=jax.ShapeDtypeStruct((S, D), x.dtype),
    )(xf, pe)

    return jnp.concatenate([part1, part2], axis=0).reshape(B, S, D)
